# baseline (speedup 1.0000x reference)
import functools

import jax
import jax.numpy as jnp
from jax import lax
from jax.experimental import pallas as pl
from jax.experimental.pallas import tpu as pltpu
from jax.experimental.pallas import tpu_sc as plsc

_NUM_WORKERS = 32
_NBUF = 4


def _plan(num_rows):
    assert num_rows % 8 == 0, num_rows
    granules = num_rows // 8
    lo = granules // _NUM_WORKERS
    nbig = granules - lo * _NUM_WORKERS
    small = lo * 8
    big = small + 8
    chunk = 8
    for c in range(min(small, 208), 0, -8):
        if small % c == 0:
            chunk = c
            break
    return big, small, nbig, chunk, small // chunk


@functools.lru_cache(maxsize=None)
def _make(num_rows, num_types, dim):
    big, small, nbig, chunk, nchunks = _plan(num_rows)
    mesh = plsc.VectorSubcoreMesh(core_axis_name="c", subcore_axis_name="s")
    tab_shards = 1
    for s in range(16, 0, -1):
        if num_types % s == 0 and (num_types // s) % 8 == 0:
            tab_shards = s
            break
    tab_rows = num_types // tab_shards

    @functools.partial(
        pl.kernel,
        mesh=mesh,
        out_type=jax.ShapeDtypeStruct((num_rows, dim), jnp.float32),
        scratch_types=(
            [pltpu.VMEM_SHARED((num_types, dim), jnp.float32),
             pltpu.VMEM((big,), jnp.int32)]
            + [pltpu.VMEM((chunk, dim), jnp.float32) for _ in range(_NBUF)]
            + [pltpu.VMEM((8, dim), jnp.float32)]
            + [pltpu.SemaphoreType.DMA for _ in range(2 * _NBUF)]
        ),
    )
    def gather_kernel(idx_hbm, table_hbm, out_hbm, table_sh, idx_v, *rest):
        rows = rest[:_NBUF]
        rows_t = rest[_NBUF]
        gsems = rest[_NBUF + 1:2 * _NBUF + 1]
        ssems = rest[2 * _NBUF + 1:]
        cid = lax.axis_index("c")
        sid = lax.axis_index("s")
        wid = sid * 2 + cid
        is_big = wid < nbig
        base = jnp.where(is_big, wid * big,
                         nbig * big + (wid - nbig) * small)
        base = pl.multiple_of(base, 8)

        if nbig:
            @pl.when(is_big)
            def _():
                pltpu.sync_copy(idx_hbm.at[pl.ds(base, big)], idx_v)

            @pl.when(jnp.logical_not(is_big))
            def _():
                pltpu.sync_copy(idx_hbm.at[pl.ds(base, small)],
                                idx_v.at[pl.ds(0, small)])
        else:
            pltpu.sync_copy(idx_hbm.at[pl.ds(base, small)],
                            idx_v.at[pl.ds(0, small)])

        @pl.when(sid < tab_shards)
        def _():
            shard = sid * tab_rows
            pltpu.sync_copy(table_hbm.at[pl.ds(shard, tab_rows), :],
                            table_sh.at[pl.ds(shard, tab_rows), :])
        plsc.subcore_barrier()

        def gather(j, b):
            return pltpu.async_copy(
                table_sh.at[idx_v.at[pl.ds(j * chunk, chunk)]],
                rows[b], gsems[b])

        gathers = [None] * _NBUF
        scatters = [None] * _NBUF
        for p in range(min(_NBUF - 1, nchunks)):
            gathers[p] = gather(p, p)
        for j in range(nchunks):
            b = j % _NBUF
            gathers[j % _NBUF].wait()
            nxt = j + _NBUF - 1
            if nxt < nchunks:
                nb = nxt % _NBUF
                if scatters[nb] is not None:
                    scatters[nb].wait()
                gathers[nb] = gather(nxt, nb)
            start = pl.multiple_of(base + j * chunk, 8)
            scatters[b] = pltpu.async_copy(
                rows[b], out_hbm.at[pl.ds(start, chunk), :], ssems[b])

        if nbig:
            @pl.when(is_big)
            def _tail():
                pltpu.async_copy(
                    table_sh.at[idx_v.at[pl.ds(nchunks * chunk, 8)]],
                    rows_t, gsems[0]).wait()
                start = pl.multiple_of(base + nchunks * chunk, 8)
                pltpu.sync_copy(rows_t, out_hbm.at[pl.ds(start, 8), :])

        for s in scatters:
            if s is not None:
                s.wait()

    return gather_kernel


def kernel(node_types, type_embedding):
    (num_rows,) = node_types.shape
    num_types, dim = type_embedding.shape
    idx = node_types.astype(jnp.int32)
    table = type_embedding.astype(jnp.float32)
    return _make(num_rows, num_types, dim)(idx, table)


# speedup vs baseline: 1.0179x; 1.0179x over previous
import functools

import jax
import jax.numpy as jnp
from jax import lax
from jax.experimental import pallas as pl
from jax.experimental.pallas import tpu as pltpu
from jax.experimental.pallas import tpu_sc as plsc

_NUM_WORKERS = 32
_NBUF = 3


def _plan(num_rows):
    assert num_rows % 8 == 0, num_rows
    granules = num_rows // 8
    lo = granules // _NUM_WORKERS
    nbig = granules - lo * _NUM_WORKERS
    small = lo * 8
    big = small + 8
    chunk = 8
    for c in range(min(small, 304), 0, -8):
        if small % c == 0:
            chunk = c
            break
    return big, small, nbig, chunk, small // chunk


@functools.lru_cache(maxsize=None)
def _make(num_rows, num_types, dim):
    big, small, nbig, chunk, nchunks = _plan(num_rows)
    mesh = plsc.VectorSubcoreMesh(core_axis_name="c", subcore_axis_name="s")
    tab_shards = 1
    for s in range(16, 0, -1):
        if num_types % s == 0 and (num_types // s) % 8 == 0:
            tab_shards = s
            break
    tab_rows = num_types // tab_shards

    @functools.partial(
        pl.kernel,
        mesh=mesh,
        out_type=jax.ShapeDtypeStruct((num_rows, dim), jnp.float32),
        scratch_types=(
            [pltpu.VMEM_SHARED((num_types, dim), jnp.float32),
             pltpu.VMEM((big,), jnp.int32)]
            + [pltpu.VMEM((chunk, dim), jnp.float32) for _ in range(_NBUF)]
            + [pltpu.VMEM((8, dim), jnp.float32)]
            + [pltpu.SemaphoreType.DMA for _ in range(2 * _NBUF + 1)]
        ),
    )
    def gather_kernel(idx_hbm, table_hbm, out_hbm, table_sh, idx_v, *rest):
        rows = rest[:_NBUF]
        rows_t = rest[_NBUF]
        gsems = rest[_NBUF + 1:2 * _NBUF + 1]
        ssems = rest[2 * _NBUF + 1:3 * _NBUF + 1]
        tsem = rest[3 * _NBUF + 1]
        cid = lax.axis_index("c")
        sid = lax.axis_index("s")
        wid = sid * 2 + cid
        is_big = wid < nbig
        base = jnp.where(is_big, wid * big,
                         nbig * big + (wid - nbig) * small)
        base = pl.multiple_of(base, 8)

        if nbig:
            @pl.when(is_big)
            def _():
                pltpu.sync_copy(idx_hbm.at[pl.ds(base, big)], idx_v)

            @pl.when(jnp.logical_not(is_big))
            def _():
                pltpu.sync_copy(idx_hbm.at[pl.ds(base, small)],
                                idx_v.at[pl.ds(0, small)])
        else:
            pltpu.sync_copy(idx_hbm.at[pl.ds(base, small)],
                            idx_v.at[pl.ds(0, small)])

        @pl.when(sid < tab_shards)
        def _():
            shard = sid * tab_rows
            pltpu.sync_copy(table_hbm.at[pl.ds(shard, tab_rows), :],
                            table_sh.at[pl.ds(shard, tab_rows), :])
        plsc.subcore_barrier()

        def gather(j, b):
            return pltpu.async_copy(
                table_sh.at[idx_v.at[pl.ds(j * chunk, chunk)]],
                rows[b], gsems[b])

        if nbig:
            @pl.when(is_big)
            def _():
                pltpu.async_copy(
                    table_sh.at[idx_v.at[pl.ds(nchunks * chunk, 8)]],
                    rows_t, tsem)

        gathers = [None] * _NBUF
        scatters = [None] * _NBUF
        gathers[0] = gather(0, 0)
        if nchunks > 1:
            gathers[1] = gather(1, 1)
        for j in range(nchunks):
            b = j % _NBUF
            gathers[j % _NBUF].wait()
            nxt = j + 2
            if nxt < nchunks:
                nb = nxt % _NBUF
                if scatters[nb] is not None:
                    scatters[nb].wait()
                gathers[nb] = gather(nxt, nb)
            start = pl.multiple_of(base + j * chunk, 8)
            scatters[b] = pltpu.async_copy(
                rows[b], out_hbm.at[pl.ds(start, chunk), :], ssems[b])

        if nbig:
            @pl.when(is_big)
            def _tail():
                pltpu.make_async_copy(out_hbm.at[pl.ds(0, 8), :], rows_t,
                                      tsem).wait()
                start = pl.multiple_of(base + nchunks * chunk, 8)
                pltpu.sync_copy(rows_t, out_hbm.at[pl.ds(start, 8), :])

        for s in scatters:
            if s is not None:
                s.wait()

    return gather_kernel


def kernel(node_types, type_embedding):
    (num_rows,) = node_types.shape
    num_types, dim = type_embedding.shape
    idx = node_types.astype(jnp.int32)
    table = type_embedding.astype(jnp.float32)
    return _make(num_rows, num_types, dim)(idx, table)


# speedup vs baseline: 1.0216x; 1.0036x over previous
import functools

import jax
import jax.numpy as jnp
from jax import lax
from jax.experimental import pallas as pl
from jax.experimental.pallas import tpu as pltpu
from jax.experimental.pallas import tpu_sc as plsc

_NUM_WORKERS = 32
_NBUF = 3


def _plan(num_rows):
    assert num_rows % 8 == 0, num_rows
    granules = num_rows // 8
    lo = granules // _NUM_WORKERS
    nbig = granules - lo * _NUM_WORKERS
    small = lo * 8
    big = small + 8
    chunk = 8
    for c in range(min(small, 304), 0, -8):
        if small % c == 0:
            chunk = c
            break
    return big, small, nbig, chunk, small // chunk


@functools.lru_cache(maxsize=None)
def _make(num_rows, num_types, dim):
    big, small, nbig, chunk, nchunks = _plan(num_rows)
    mesh = plsc.VectorSubcoreMesh(core_axis_name="c", subcore_axis_name="s")
    tab_shards = 1
    for s in range(16, 0, -1):
        if num_types % s == 0 and (num_types // s) % 8 == 0:
            tab_shards = s
            break
    tab_rows = num_types // tab_shards

    @functools.partial(
        pl.kernel,
        mesh=mesh,
        out_type=jax.ShapeDtypeStruct((num_rows, dim), jnp.float32),
        scratch_types=(
            [pltpu.VMEM_SHARED((num_types, dim), jnp.float32),
             pltpu.VMEM((big,), jnp.int32)]
            + [pltpu.VMEM((chunk, dim), jnp.float32) for _ in range(_NBUF)]
            + [pltpu.VMEM((8, dim), jnp.float32)]
            + [pltpu.SemaphoreType.DMA for _ in range(2 * _NBUF + 1)]
        ),
    )
    def gather_kernel(idx_hbm, table_hbm, out_hbm, table_sh, idx_v, *rest):
        rows = rest[:_NBUF]
        rows_t = rest[_NBUF]
        gsems = rest[_NBUF + 1:2 * _NBUF + 1]
        ssems = rest[2 * _NBUF + 1:3 * _NBUF + 1]
        tsem = rest[3 * _NBUF + 1]
        cid = lax.axis_index("c")
        sid = lax.axis_index("s")
        wid = sid * 2 + cid
        is_big = wid < nbig
        base = jnp.where(is_big, wid * big,
                         nbig * big + (wid - nbig) * small)
        base = pl.multiple_of(base, 8)

        if nbig:
            @pl.when(is_big)
            def _():
                pltpu.sync_copy(idx_hbm.at[pl.ds(base, big)], idx_v)

            @pl.when(jnp.logical_not(is_big))
            def _():
                pltpu.sync_copy(idx_hbm.at[pl.ds(base, small)],
                                idx_v.at[pl.ds(0, small)])
        else:
            pltpu.sync_copy(idx_hbm.at[pl.ds(base, small)],
                            idx_v.at[pl.ds(0, small)])

        @pl.when(sid < tab_shards)
        def _():
            shard = sid * tab_rows
            pltpu.sync_copy(table_hbm.at[pl.ds(shard, tab_rows), :],
                            table_sh.at[pl.ds(shard, tab_rows), :])
        plsc.subcore_barrier()

        def gather(j, b):
            return pltpu.async_copy(
                table_sh.at[idx_v.at[pl.ds(j * chunk, chunk)]],
                rows[b], gsems[b])

        if nbig:
            @pl.when(is_big)
            def _():
                pltpu.async_copy(
                    table_sh.at[idx_v.at[pl.ds(nchunks * chunk, 8)]],
                    rows_t, tsem)

        gathers = [None] * _NBUF
        scatters = [None] * _NBUF
        gathers[0] = gather(0, 0)
        if nchunks > 1:
            gathers[1] = gather(1, 1)
        for j in range(nchunks):
            b = j % _NBUF
            gathers[j % _NBUF].wait()
            nxt = j + 2
            if nxt < nchunks:
                nb = nxt % _NBUF
                if scatters[nb] is not None:
                    scatters[nb].wait()
                gathers[nb] = gather(nxt, nb)
            start = pl.multiple_of(base + j * chunk, 8)
            scatters[b] = pltpu.async_copy(
                rows[b], out_hbm.at[pl.ds(start, chunk), :], ssems[b])

        if nbig:
            @pl.when(is_big)
            def _tail():
                pltpu.make_async_copy(out_hbm.at[pl.ds(0, 8), :], rows_t,
                                      tsem).wait()
                start = pl.multiple_of(base + nchunks * chunk, 8)
                pltpu.async_copy(rows_t, out_hbm.at[pl.ds(start, 8), :],
                                 tsem)

        for s in scatters:
            if s is not None:
                s.wait()

        if nbig:
            @pl.when(is_big)
            def _():
                pltpu.make_async_copy(rows_t, out_hbm.at[pl.ds(0, 8), :],
                                      tsem).wait()

    return gather_kernel


def kernel(node_types, type_embedding):
    (num_rows,) = node_types.shape
    num_types, dim = type_embedding.shape
    idx = node_types.astype(jnp.int32)
    table = type_embedding.astype(jnp.float32)
    return _make(num_rows, num_types, dim)(idx, table)
